# baseline (device time: 59181 ns/iter reference)
import jax
import jax.numpy as jnp
from jax import lax
from jax.experimental import pallas as pl
from jax.experimental.pallas import tpu as pltpu

N_DEV = 4
SQ = 256
D_MODEL = 1024
DH = 128
H_LOC = 8
SCALE = 0.08838834764831843


def kernel(x, Wq, Wo, Wk, Wv):
    def body(x_ref, wq_ref, wo_ref, wk_ref, wv_ref, out_ref,
             comm_ref, send_sems, recv_sems):
        my_pos = lax.axis_index("i")
        left = (my_pos - 1) % N_DEV
        right = (my_pos + 1) % N_DEV

        barrier_sem = pltpu.get_barrier_semaphore()
        for nbr in (left, right):
            pl.semaphore_signal(
                barrier_sem, inc=1,
                device_id=(nbr,), device_id_type=pl.DeviceIdType.MESH,
            )
        pl.semaphore_wait(barrier_sem, 2)

        xm = x_ref[0]
        q = jnp.dot(xm, wq_ref[...], preferred_element_type=jnp.float32)
        k = jnp.dot(xm, wk_ref[...], preferred_element_type=jnp.float32)
        v = jnp.dot(xm, wv_ref[...], preferred_element_type=jnp.float32)

        partial = jnp.zeros((SQ, D_MODEL), jnp.float32)
        for h in range(H_LOC):
            qh = q[:, h * DH:(h + 1) * DH]
            kh = k[:, h * DH:(h + 1) * DH]
            vh = v[:, h * DH:(h + 1) * DH]
            s = lax.dot_general(
                qh, kh, (((1,), (1,)), ((), ())),
                preferred_element_type=jnp.float32,
            ) * SCALE
            m = jnp.max(s, axis=1, keepdims=True)
            p = jnp.exp(s - m)
            l = jnp.sum(p, axis=1, keepdims=True)
            o = jnp.dot(p, vh, preferred_element_type=jnp.float32) / l
            partial = partial + jnp.dot(
                o, wo_ref[h * DH:(h + 1) * DH, :],
                preferred_element_type=jnp.float32,
            )

        comm_ref[0] = partial

        for hop in range(N_DEV - 1):
            rdma = pltpu.make_async_remote_copy(
                src_ref=comm_ref.at[hop],
                dst_ref=comm_ref.at[hop + 1],
                send_sem=send_sems.at[hop],
                recv_sem=recv_sems.at[hop],
                device_id=(right,),
                device_id_type=pl.DeviceIdType.MESH,
            )
            rdma.start()
            rdma.wait()

        out_ref[0] = comm_ref[0] + comm_ref[1] + comm_ref[2] + comm_ref[3]

    return pl.pallas_call(
        body,
        out_shape=jax.ShapeDtypeStruct((1, SQ, D_MODEL), jnp.float32),
        in_specs=[pl.BlockSpec(memory_space=pltpu.VMEM)] * 5,
        out_specs=pl.BlockSpec(memory_space=pltpu.VMEM),
        scratch_shapes=[
            pltpu.VMEM((N_DEV, SQ, D_MODEL), jnp.float32),
            pltpu.SemaphoreType.DMA((N_DEV - 1,)),
            pltpu.SemaphoreType.DMA((N_DEV - 1,)),
        ],
        compiler_params=pltpu.CompilerParams(collective_id=0),
    )(x, Wq, Wo, Wk, Wv)


# device time: 35468 ns/iter; 1.6686x vs baseline; 1.6686x over previous
import jax
import jax.numpy as jnp
from jax import lax
from jax.experimental import pallas as pl
from jax.experimental.pallas import tpu as pltpu

N_DEV = 4
SQ = 256
RPC = SQ // N_DEV
D_MODEL = 1024
DH = 128
H_LOC = 8
SCALE = 0.08838834764831843


def kernel(x, Wq, Wo, Wk, Wv):
    def body(x_ref, wq_ref, wo_ref, wk_ref, wv_ref, out_ref,
             part_ref, rs_ref, own_ref,
             rs_send, rs_recv, ag_send, ag_recv):
        d = lax.axis_index("i")

        barrier_sem = pltpu.get_barrier_semaphore()
        for o in (1, 2, 3):
            pl.semaphore_signal(
                barrier_sem, inc=1,
                device_id=((d + o) % N_DEV,),
                device_id_type=pl.DeviceIdType.MESH,
            )
        pl.semaphore_wait(barrier_sem, 3)

        xm = x_ref[0]
        q = jnp.dot(xm, wq_ref[...], preferred_element_type=jnp.float32)
        k = jnp.dot(xm, wk_ref[...], preferred_element_type=jnp.float32)
        v = jnp.dot(xm, wv_ref[...], preferred_element_type=jnp.float32)

        partial = jnp.zeros((SQ, D_MODEL), jnp.float32)
        for h in range(H_LOC):
            qh = q[:, h * DH:(h + 1) * DH]
            kh = k[:, h * DH:(h + 1) * DH]
            vh = v[:, h * DH:(h + 1) * DH]
            s = lax.dot_general(
                qh, kh, (((1,), (1,)), ((), ())),
                preferred_element_type=jnp.float32,
            ) * SCALE
            m = jnp.max(s, axis=1, keepdims=True)
            p = jnp.exp(s - m)
            l = jnp.sum(p, axis=1, keepdims=True)
            o_h = jnp.dot(p, vh, preferred_element_type=jnp.float32) / l
            partial = partial + jnp.dot(
                o_h, wo_ref[h * DH:(h + 1) * DH, :],
                preferred_element_type=jnp.float32,
            )

        part_ref[...] = partial.reshape(N_DEV, RPC, D_MODEL)

        rs_rdmas = []
        for o in (1, 2, 3):
            t = (d + o) % N_DEV
            r = pltpu.make_async_remote_copy(
                src_ref=part_ref.at[t],
                dst_ref=rs_ref.at[o - 1],
                send_sem=rs_send.at[o - 1],
                recv_sem=rs_recv.at[o - 1],
                device_id=(t,),
                device_id_type=pl.DeviceIdType.MESH,
            )
            r.start()
            rs_rdmas.append(r)
        for r in rs_rdmas:
            r.wait_recv()

        own = part_ref[d] + rs_ref[0] + rs_ref[1] + rs_ref[2]
        own_ref[...] = own
        out_ref[0, pl.ds(d * RPC, RPC), :] = own

        ag_rdmas = []
        for o in (1, 2, 3):
            t = (d + o) % N_DEV
            r = pltpu.make_async_remote_copy(
                src_ref=own_ref,
                dst_ref=out_ref.at[0, pl.ds(d * RPC, RPC), :],
                send_sem=ag_send.at[o - 1],
                recv_sem=ag_recv.at[o - 1],
                device_id=(t,),
                device_id_type=pl.DeviceIdType.MESH,
            )
            r.start()
            ag_rdmas.append(r)
        for r in ag_rdmas:
            r.wait_recv()
        for r in rs_rdmas:
            r.wait_send()
        for r in ag_rdmas:
            r.wait_send()

    return pl.pallas_call(
        body,
        out_shape=jax.ShapeDtypeStruct((1, SQ, D_MODEL), jnp.float32),
        in_specs=[pl.BlockSpec(memory_space=pltpu.VMEM)] * 5,
        out_specs=pl.BlockSpec(memory_space=pltpu.VMEM),
        scratch_shapes=[
            pltpu.VMEM((N_DEV, RPC, D_MODEL), jnp.float32),
            pltpu.VMEM((N_DEV - 1, RPC, D_MODEL), jnp.float32),
            pltpu.VMEM((RPC, D_MODEL), jnp.float32),
            pltpu.SemaphoreType.DMA((N_DEV - 1,)),
            pltpu.SemaphoreType.DMA((N_DEV - 1,)),
            pltpu.SemaphoreType.DMA((N_DEV - 1,)),
            pltpu.SemaphoreType.DMA((N_DEV - 1,)),
        ],
        compiler_params=pltpu.CompilerParams(collective_id=0),
    )(x, Wq, Wo, Wk, Wv)


# device time: 33569 ns/iter; 1.7630x vs baseline; 1.0566x over previous
import jax
import jax.numpy as jnp
from jax import lax
from jax.experimental import pallas as pl
from jax.experimental.pallas import tpu as pltpu

N_DEV = 4
SQ = 256
RPC = SQ // N_DEV
D_MODEL = 1024
DH = 128
H_LOC = 8
SCALE = 0.08838834764831843


def kernel(x, Wq, Wo, Wk, Wv):
    def body(x_ref, wq_ref, wo_ref, wk_ref, wv_ref, out_ref,
             q_ref, send_ref, rs_ref, own_ref, ag_ref,
             rs_send, rs_recv, ag_send, ag_recv):
        d = lax.axis_index("i")

        barrier_sem = pltpu.get_barrier_semaphore()
        for o in (1, 2, 3):
            pl.semaphore_signal(
                barrier_sem, inc=1,
                device_id=((d + o) % N_DEV,),
                device_id_type=pl.DeviceIdType.MESH,
            )
        pl.semaphore_wait(barrier_sem, 3)

        xm = x_ref[0]
        q = jnp.dot(xm, wq_ref[...], preferred_element_type=jnp.float32)
        k = jnp.dot(xm, wk_ref[...], preferred_element_type=jnp.float32)
        v = jnp.dot(xm, wv_ref[...], preferred_element_type=jnp.float32)
        q_ref[...] = q.reshape(N_DEV, RPC, D_MODEL)

        def partial_rows(t):
            qt = q_ref[t]
            acc = jnp.zeros((RPC, D_MODEL), jnp.float32)
            for h in range(H_LOC):
                qh = qt[:, h * DH:(h + 1) * DH]
                kh = k[:, h * DH:(h + 1) * DH]
                vh = v[:, h * DH:(h + 1) * DH]
                s = lax.dot_general(
                    qh, kh, (((1,), (1,)), ((), ())),
                    preferred_element_type=jnp.float32,
                ) * SCALE
                m = jnp.max(s, axis=1, keepdims=True)
                p = jnp.exp(s - m)
                l = jnp.sum(p, axis=1, keepdims=True)
                o_h = jnp.dot(p, vh, preferred_element_type=jnp.float32) / l
                acc = acc + jnp.dot(
                    o_h, wo_ref[h * DH:(h + 1) * DH, :],
                    preferred_element_type=jnp.float32,
                )
            return acc

        rs_rdmas = []
        for j, o in enumerate((1, 2, 3)):
            t = (d + o) % N_DEV
            send_ref[j] = partial_rows(t).astype(jnp.bfloat16)
            r = pltpu.make_async_remote_copy(
                src_ref=send_ref.at[j],
                dst_ref=rs_ref.at[o - 1],
                send_sem=rs_send.at[o - 1],
                recv_sem=rs_recv.at[o - 1],
                device_id=(t,),
                device_id_type=pl.DeviceIdType.MESH,
            )
            r.start()
            rs_rdmas.append(r)

        own_part = partial_rows(d)

        for r in rs_rdmas:
            r.wait_recv()
        own = (own_part
               + rs_ref[0].astype(jnp.float32)
               + rs_ref[1].astype(jnp.float32)
               + rs_ref[2].astype(jnp.float32))
        out_ref[0, pl.ds(d * RPC, RPC), :] = own
        own_ref[...] = own.astype(jnp.bfloat16)

        ag_rdmas = []
        for o in (1, 2, 3):
            t = (d + o) % N_DEV
            r = pltpu.make_async_remote_copy(
                src_ref=own_ref,
                dst_ref=ag_ref.at[o - 1],
                send_sem=ag_send.at[o - 1],
                recv_sem=ag_recv.at[o - 1],
                device_id=(t,),
                device_id_type=pl.DeviceIdType.MESH,
            )
            r.start()
            ag_rdmas.append(r)
        for o in (1, 2, 3):
            ag_rdmas[o - 1].wait_recv()
            src = (d - o) % N_DEV
            out_ref[0, pl.ds(src * RPC, RPC), :] = (
                ag_ref[o - 1].astype(jnp.float32))

        for r in rs_rdmas:
            r.wait_send()
        for r in ag_rdmas:
            r.wait_send()

    return pl.pallas_call(
        body,
        out_shape=jax.ShapeDtypeStruct((1, SQ, D_MODEL), jnp.float32),
        in_specs=[pl.BlockSpec(memory_space=pltpu.VMEM)] * 5,
        out_specs=pl.BlockSpec(memory_space=pltpu.VMEM),
        scratch_shapes=[
            pltpu.VMEM((N_DEV, RPC, D_MODEL), jnp.float32),
            pltpu.VMEM((N_DEV - 1, RPC, D_MODEL), jnp.bfloat16),
            pltpu.VMEM((N_DEV - 1, RPC, D_MODEL), jnp.bfloat16),
            pltpu.VMEM((RPC, D_MODEL), jnp.bfloat16),
            pltpu.VMEM((N_DEV - 1, RPC, D_MODEL), jnp.bfloat16),
            pltpu.SemaphoreType.DMA((N_DEV - 1,)),
            pltpu.SemaphoreType.DMA((N_DEV - 1,)),
            pltpu.SemaphoreType.DMA((N_DEV - 1,)),
            pltpu.SemaphoreType.DMA((N_DEV - 1,)),
        ],
        compiler_params=pltpu.CompilerParams(collective_id=0),
    )(x, Wq, Wo, Wk, Wv)


# device time: 27287 ns/iter; 2.1688x vs baseline; 1.2302x over previous
import functools

import jax
import jax.numpy as jnp
from jax import lax
from jax.experimental import pallas as pl
from jax.experimental.pallas import tpu as pltpu

N_DEV = 4
SQ = 256
D_MODEL = 1024
CPC = D_MODEL // N_DEV
DH = 128
H_LOC = 8
SCALE = 0.08838834764831843
F32 = jnp.float32
BF16 = jnp.bfloat16


def kernel(x, Wq, Wo, Wk, Wv):
    def body(x_ref, wq_ref, wo_ref, wk_ref, wv_ref, out_ref,
             send_ref, rs_ref, own_ref, ownb_ref, ag_ref,
             rs_send, rs_recv, ag_send, ag_recv):
        d = lax.axis_index("i")

        barrier_sem = pltpu.get_barrier_semaphore()
        for o in (1, 2, 3):
            pl.semaphore_signal(
                barrier_sem, inc=1,
                device_id=((d + o) % N_DEV,),
                device_id_type=pl.DeviceIdType.MESH,
            )
        pl.semaphore_wait(barrier_sem, 3)

        xm = x_ref[0]
        q = jnp.dot(xm, wq_ref[...], preferred_element_type=F32)
        k = jnp.dot(xm, wk_ref[...], preferred_element_type=F32)
        v = jnp.dot(xm, wv_ref[...], preferred_element_type=F32)

        o_heads = []
        for h in range(H_LOC):
            qh = q[:, h * DH:(h + 1) * DH]
            kh = k[:, h * DH:(h + 1) * DH]
            vh = v[:, h * DH:(h + 1) * DH]
            s = lax.dot_general(
                qh, kh, (((1,), (1,)), ((), ())),
                preferred_element_type=F32,
            ) * SCALE
            p = jnp.exp(s)
            l = jnp.sum(p, axis=1, keepdims=True)
            o_heads.append(jnp.dot(p, vh, preferred_element_type=F32) / l)
        attn = jnp.concatenate(o_heads, axis=1)

        for c in range(N_DEV):
            chunk = jnp.dot(attn, wo_ref[:, c * CPC:(c + 1) * CPC],
                            preferred_element_type=F32)

            @pl.when(c != d)
            def _(c=c, chunk=chunk):
                send_ref[c] = chunk.astype(BF16)
                slot = (c - d) % N_DEV - 1
                pltpu.make_async_remote_copy(
                    src_ref=send_ref.at[c],
                    dst_ref=rs_ref.at[slot],
                    send_sem=rs_send.at[slot],
                    recv_sem=rs_recv.at[slot],
                    device_id=(c,),
                    device_id_type=pl.DeviceIdType.MESH,
                ).start()

            @pl.when(c == d)
            def _(chunk=chunk):
                own_ref[...] = chunk

        def waiter(dst, sem):
            return pltpu.make_async_remote_copy(
                src_ref=send_ref.at[0], dst_ref=dst,
                send_sem=sem, recv_sem=sem,
                device_id=(d,), device_id_type=pl.DeviceIdType.MESH,
            )

        for j in range(N_DEV - 1):
            waiter(rs_ref.at[j], rs_recv.at[j]).wait_recv()
        own = (own_ref[...]
               + rs_ref[0].astype(F32)
               + rs_ref[1].astype(F32)
               + rs_ref[2].astype(F32))
        ownb_ref[...] = own.astype(BF16)

        ag_rdmas = []
        for o in (1, 2, 3):
            r = pltpu.make_async_remote_copy(
                src_ref=ownb_ref,
                dst_ref=ag_ref.at[o - 1],
                send_sem=ag_send.at[o - 1],
                recv_sem=ag_recv.at[o - 1],
                device_id=((d + o) % N_DEV,),
                device_id_type=pl.DeviceIdType.MESH,
            )
            r.start()
            ag_rdmas.append(r)

        for c in range(N_DEV):
            @pl.when(c == d)
            def _(c=c):
                out_ref[0, :, c * CPC:(c + 1) * CPC] = own

        for r in ag_rdmas:
            r.wait_recv()
        for c in range(N_DEV):
            @pl.when(c != d)
            def _(c=c):
                j = (d - c) % N_DEV - 1
                out_ref[0, :, c * CPC:(c + 1) * CPC] = ag_ref[j].astype(F32)

        for j in range(N_DEV - 1):
            waiter(rs_ref.at[j], rs_send.at[j]).wait_send()
        for r in ag_rdmas:
            r.wait_send()

    return pl.pallas_call(
        body,
        out_shape=jax.ShapeDtypeStruct((1, SQ, D_MODEL), F32),
        in_specs=[pl.BlockSpec(memory_space=pltpu.VMEM)] * 5,
        out_specs=pl.BlockSpec(memory_space=pltpu.VMEM),
        scratch_shapes=[
            pltpu.VMEM((N_DEV, SQ, CPC), BF16),
            pltpu.VMEM((N_DEV - 1, SQ, CPC), BF16),
            pltpu.VMEM((SQ, CPC), F32),
            pltpu.VMEM((SQ, CPC), BF16),
            pltpu.VMEM((N_DEV - 1, SQ, CPC), BF16),
            pltpu.SemaphoreType.DMA((N_DEV - 1,)),
            pltpu.SemaphoreType.DMA((N_DEV - 1,)),
            pltpu.SemaphoreType.DMA((N_DEV - 1,)),
            pltpu.SemaphoreType.DMA((N_DEV - 1,)),
        ],
        compiler_params=pltpu.CompilerParams(collective_id=0),
    )(x, Wq, Wo, Wk, Wv)
